# Initial kernel scaffold; baseline (speedup 1.0000x reference)
import jax
import jax.numpy as jnp
from jax.experimental import pallas as pl


def kernel(logits, hidden_states, W1, b1, W2, b2, temperature):
    raise NotImplementedError("write your pallas kernel here")


# speedup vs baseline: 54.6162x; 54.6162x over previous
import functools

import jax
import jax.numpy as jnp
from jax.experimental import pallas as pl
from jax.experimental.pallas import tpu as pltpu

_B, _V, _S, _D = 64, 100000, 32, 768
_LANES = 128
_VP = ((_V + _LANES - 1) // _LANES) * _LANES
_R = 8
_IMIN = -2147483648
_KEY_NEG_INF = -2139095040
_KEY_POS_INF = 2139095040
_TOPK = 50
_P = 0.9


def _order_key(x):
    b = jax.lax.bitcast_convert_type(x, jnp.int32)
    return jnp.where(b >= 0, b, jnp.int32(_IMIN) - b)


def _midpoint(lo, hi):
    return (lo >> 1) + (hi >> 1) + (lo & hi & 1)


def _body(t_ref, l_ref, g_ref, h_ref, w1_ref, b1_ref, w2_ref, b2_ref,
          out_ref, e_ref, key_ref):
    t = t_ref[0, 0]
    l = l_ref[...] / t
    key = _order_key(l)
    key_ref[...] = key
    m = jnp.max(l, axis=-1, keepdims=True)
    e = jnp.exp(l - m)
    e_ref[...] = e
    z = jnp.sum(e, axis=-1, keepdims=True)
    pz = jnp.float32(_P) * z

    ones = jnp.ones((_R, 1), dtype=jnp.int32)
    lo0 = ones * _KEY_NEG_INF
    hi0 = ones * _KEY_POS_INF

    def it(_, carry):
        lo_k, hi_k, lo_n, hi_n = carry
        mid_k = _midpoint(lo_k, hi_k)
        mid_n = _midpoint(lo_n, hi_n)
        kk = key_ref[...]
        cnt = jnp.sum(jnp.where(kk > mid_k, jnp.float32(1.0), jnp.float32(0.0)),
                      axis=-1, keepdims=True)
        gs = jnp.sum(jnp.where(kk > mid_n, e_ref[...], jnp.float32(0.0)),
                     axis=-1, keepdims=True)
        big_k = cnt >= jnp.float32(_TOPK)
        lo_k = jnp.where(big_k, mid_k, lo_k)
        hi_k = jnp.where(big_k, hi_k, mid_k)
        big_n = gs > pz
        lo_n = jnp.where(big_n, mid_n, lo_n)
        hi_n = jnp.where(big_n, hi_n, mid_n)
        return lo_k, hi_k, lo_n, hi_n

    lo_k, _, lo_n, _ = jax.lax.fori_loop(0, 32, it, (lo0, hi0, lo0, hi0))

    keep_k = key > lo_k
    keep_n = key > lo_n

    probs = e / z
    logp = jnp.log(probs + jnp.float32(1e-10))
    ent = -jnp.sum(probs * logp, axis=-1, keepdims=True)
    keep_y = jnp.abs(-logp - ent) < jnp.float32(0.5)

    neg_inf = jnp.float32(-jnp.inf)
    iota = jax.lax.broadcasted_iota(jnp.int32, (_R, _VP), 1)
    sentinel = jnp.int32(_VP)

    def sample(keep, s):
        vals = jnp.where(keep, l, neg_inf) + g_ref[s]
        mx = jnp.max(vals, axis=-1, keepdims=True)
        return jnp.min(jnp.where(vals == mx, iota, sentinel),
                       axis=-1, keepdims=True)

    s_n = sample(keep_n, 0)
    s_k = sample(keep_k, 1)
    s_t = sample(jnp.ones((_R, _VP), dtype=jnp.bool_), 2)
    s_y = sample(keep_y, 3)
    samples = jnp.concatenate([s_n, s_k, s_t, s_y], axis=-1).astype(jnp.float32)

    h = jnp.mean(h_ref[...], axis=1)
    z1 = jax.nn.relu(
        jnp.dot(h, w1_ref[...], preferred_element_type=jnp.float32)
        + b1_ref[...])
    z2 = (jnp.dot(z1, w2_ref[...], preferred_element_type=jnp.float32)
          + b2_ref[...])
    w = jax.nn.softmax(z2, axis=-1)
    weighted = jnp.sum(samples * w, axis=-1, keepdims=True)
    out_ref[...] = weighted.astype(jnp.int32)


@functools.partial(jax.jit, static_argnames=())
def kernel(logits, hidden_states, W1, b1, W2, b2, temperature=1.0):
    lp = jnp.pad(logits, ((0, 0), (0, _VP - _V)),
                 constant_values=-jnp.inf)
    skey = jax.random.key(42)
    g = jnp.stack([
        jax.random.gumbel(jax.random.fold_in(skey, i), (_B, _V), jnp.float32)
        for i in range(4)])
    gp = jnp.pad(g, ((0, 0), (0, 0), (0, _VP - _V)))
    t = jnp.asarray(temperature, jnp.float32).reshape(1, 1)
    b1r = b1.reshape(1, 256)
    b2r = b2.reshape(1, 4)

    grid = _B // _R
    out = pl.pallas_call(
        _body,
        grid=(grid,),
        in_specs=[
            pl.BlockSpec((1, 1), lambda i: (0, 0)),
            pl.BlockSpec((_R, _VP), lambda i: (i, 0)),
            pl.BlockSpec((4, _R, _VP), lambda i: (0, i, 0)),
            pl.BlockSpec((_R, _S, _D), lambda i: (i, 0, 0)),
            pl.BlockSpec((_D, 256), lambda i: (0, 0)),
            pl.BlockSpec((1, 256), lambda i: (0, 0)),
            pl.BlockSpec((256, 4), lambda i: (0, 0)),
            pl.BlockSpec((1, 4), lambda i: (0, 0)),
        ],
        out_specs=pl.BlockSpec((_R, 1), lambda i: (i, 0)),
        out_shape=jax.ShapeDtypeStruct((_B, 1), jnp.int32),
        scratch_shapes=[
            pltpu.VMEM((_R, _VP), jnp.float32),
            pltpu.VMEM((_R, _VP), jnp.int32),
        ],
    )(t, lp, gp, hidden_states, W1, b1r, W2, b2r)
    return out.reshape(_B)
